# baseline (speedup 1.0000x reference)
import functools

import jax
import jax.numpy as jnp
from jax import lax
from jax.experimental import pallas as pl
from jax.experimental.pallas import tpu as pltpu
from jax.experimental.pallas import tpu_sc as plsc

_ROWS = 1000000
_DIM = 64
_NC = 2
_NS = 16
_NW = _NC * _NS
_RPW = 31248
_CH = 168
_NCHW = _RPW // _CH
_TAIL = _ROWS - _NW * _RPW
_NBUF = 4
_LOOK = 2

_mesh = plsc.VectorSubcoreMesh(core_axis_name="c", subcore_axis_name="s")


@functools.partial(
    pl.kernel,
    out_type=jax.ShapeDtypeStruct((_ROWS, _DIM), jnp.float32),
    mesh=_mesh,
    compiler_params=pltpu.CompilerParams(needs_layout_passes=False),
    scratch_types=[
        pltpu.VMEM((_NBUF, _CH, _DIM), jnp.float32),
        pltpu.VMEM((16,), jnp.int32),
        pltpu.VMEM((_DIM,), jnp.float32),
        pltpu.VMEM((8, _DIM), jnp.float32),
        pltpu.SemaphoreType.DMA,
        pltpu.SemaphoreType.DMA,
    ],
)
def _sc_body(
    emb_hbm, idx_hbm, new_hbm, out_hbm, bufs, idxv, newv, tilev, in_sem, out_sem
):
    wid = lax.axis_index("s") * _NC + lax.axis_index("c")
    base = pl.multiple_of(wid * _RPW, 8)
    pltpu.sync_copy(idx_hbm, idxv)
    pltpu.sync_copy(new_hbm, newv)
    idx = jnp.max(idxv[...])

    def in_cp(c, s):
        return pltpu.make_async_copy(
            emb_hbm.at[pl.ds(base + c * _CH, _CH), :], bufs.at[s], in_sem
        )

    def out_cp(c, s):
        return pltpu.make_async_copy(
            bufs.at[s], out_hbm.at[pl.ds(base + c * _CH, _CH), :], out_sem
        )

    for c in range(min(_LOOK, _NCHW)):
        in_cp(c, c % _NBUF).start()
    for c in range(_NCHW):
        nxt = c + _LOOK
        if nxt < _NCHW:
            if nxt - _NBUF >= 0:
                out_cp(nxt - _NBUF, nxt % _NBUF).wait()
            in_cp(nxt, nxt % _NBUF).start()
        in_cp(c, c % _NBUF).wait()
        out_cp(c, c % _NBUF).start()
    for c in range(max(0, _NCHW - _NBUF), _NCHW):
        out_cp(c, c % _NBUF).wait()

    @pl.when(wid == _NW - 1)
    def _():
        t0 = pl.multiple_of(_NW * _RPW, 8)
        tcp_in = pltpu.make_async_copy(
            emb_hbm.at[pl.ds(t0, _TAIL), :], bufs.at[0].at[pl.ds(0, _TAIL), :], in_sem
        )
        tcp_in.start()
        tcp_in.wait()
        tcp_out = pltpu.make_async_copy(
            bufs.at[0].at[pl.ds(0, _TAIL), :], out_hbm.at[pl.ds(t0, _TAIL), :], out_sem
        )
        tcp_out.start()
        tcp_out.wait()

    hi = jnp.where(wid == _NW - 1, base + _RPW + _TAIL, base + _RPW)
    owns = (idx >= base) & (idx < hi)

    @pl.when(owns)
    def _():
        tile = pl.multiple_of((idx // 8) * 8, 8)
        local = idx - tile
        tin = pltpu.make_async_copy(emb_hbm.at[pl.ds(tile, 8), :], tilev, in_sem)
        tin.start()
        tin.wait()
        rows = jnp.full((16,), local, dtype=jnp.int32)
        for j in range(_DIM // 16):
            cols = lax.iota(jnp.int32, 16) + 16 * j
            plsc.store_scatter(tilev, [rows, cols], newv[pl.ds(16 * j, 16)])
        tout = pltpu.make_async_copy(tilev, out_hbm.at[pl.ds(tile, 8), :], out_sem)
        tout.start()
        tout.wait()


def kernel(embeddings, emb_index, new_emb):
    idx16 = jnp.full((16,), emb_index, dtype=jnp.int32)
    new_row = new_emb.reshape(_DIM)
    return _sc_body(embeddings, idx16, new_row)
